# baseline (speedup 1.0000x reference)
import functools
import math

import jax
import jax.numpy as jnp
from jax import lax
from jax.experimental import pallas as pl
from jax.experimental.pallas import tpu as pltpu
from jax.experimental.pallas import tpu_sc as plsc


def _pe_block(x_ref, pos_ref, o_ref, *, scale):
    o_ref[...] = x_ref[...] * scale + pos_ref[...][None, :, :]


@functools.partial(jax.jit, static_argnames=("block_s",))
def _pe_tc(x, pos_table, block_s=1024):
    batch, seq_len, d_model = x.shape
    scale = math.sqrt(float(d_model))
    grid = (seq_len // block_s,)
    return pl.pallas_call(
        functools.partial(_pe_block, scale=scale),
        grid=grid,
        in_specs=[
            pl.BlockSpec((batch, block_s, d_model), lambda s: (0, s, 0)),
            pl.BlockSpec((block_s, d_model), lambda s: (s, 0)),
        ],
        out_specs=pl.BlockSpec((batch, block_s, d_model), lambda s: (0, s, 0)),
        out_shape=jax.ShapeDtypeStruct(x.shape, x.dtype),
    )(x, pos_table)


_SC_LANES = 16


def _pe_sc_body(x_hbm, pos_hbm, o_hbm, xbuf, pbuf, *, batch, seq_len,
                d_model, rows_per_worker, chunk_rows, scale, num_cores,
                num_subcores):
    wid = lax.axis_index("s") * num_cores + lax.axis_index("c")
    row0 = wid * rows_per_worker
    chunk_elems = chunk_rows * d_model
    n_chunks = rows_per_worker // chunk_rows
    n_vec = chunk_elems // _SC_LANES
    unroll = 8

    def chunk_loop(ci, _):
        pos_off = (row0 + ci * chunk_rows) * d_model
        pltpu.sync_copy(pos_hbm.at[pl.ds(pos_off, chunk_elems)], pbuf)

        def batch_loop(b, _):
            x_off = b * (seq_len * d_model) + pos_off
            pltpu.sync_copy(x_hbm.at[pl.ds(x_off, chunk_elems)], xbuf)

            def vec_loop(i, _):
                for u in range(unroll):
                    o = (i * unroll + u) * _SC_LANES
                    xv = xbuf[pl.ds(o, _SC_LANES)]
                    pv = pbuf[pl.ds(o, _SC_LANES)]
                    xbuf[pl.ds(o, _SC_LANES)] = xv * scale + pv
                return 0

            lax.fori_loop(0, n_vec // unroll, vec_loop, 0)
            pltpu.sync_copy(xbuf, o_hbm.at[pl.ds(x_off, chunk_elems)])
            return 0

        lax.fori_loop(0, batch, batch_loop, 0)
        return 0

    lax.fori_loop(0, n_chunks, chunk_loop, 0)


@jax.jit
def _pe_sc(x, pos_table):
    batch, seq_len, d_model = x.shape
    scale = math.sqrt(float(d_model))
    info = plsc.get_sparse_core_info()
    num_cores, num_subcores = info.num_cores, info.num_subcores
    n_workers = num_cores * num_subcores
    rows_per_worker = seq_len // n_workers
    chunk_rows = 64
    chunk_elems = chunk_rows * d_model

    x_flat = x.reshape(-1)
    pos_flat = pos_table[:seq_len].reshape(-1)

    mesh = plsc.VectorSubcoreMesh(core_axis_name="c", subcore_axis_name="s")
    body = functools.partial(
        _pe_sc_body,
        batch=batch,
        seq_len=seq_len,
        d_model=d_model,
        rows_per_worker=rows_per_worker,
        chunk_rows=chunk_rows,
        scale=scale,
        num_cores=num_cores,
        num_subcores=num_subcores,
    )
    out_flat = pl.kernel(
        body,
        mesh=mesh,
        out_type=jax.ShapeDtypeStruct((batch * seq_len * d_model,), x.dtype),
        scratch_types=[
            pltpu.VMEM((chunk_elems,), jnp.float32),
            pltpu.VMEM((chunk_elems,), jnp.float32),
        ],
    )(x_flat, pos_flat)
    return out_flat.reshape(x.shape)


def kernel(x, pos_table):
    return _pe_sc(x, pos_table)


# speedup vs baseline: 1.6543x; 1.6543x over previous
import functools
import math

import jax
import jax.numpy as jnp
from jax import lax
from jax.experimental import pallas as pl
from jax.experimental.pallas import tpu as pltpu
from jax.experimental.pallas import tpu_sc as plsc


def _pe_block(x_ref, pos_ref, o_ref, *, scale):
    o_ref[...] = x_ref[...] * scale + pos_ref[...][None, :, :]


def _pe_tc(x, pos_table, tc_rows, block_s):
    batch, seq_len, d_model = x.shape
    scale = math.sqrt(float(d_model))
    grid = (tc_rows // block_s,)
    return pl.pallas_call(
        functools.partial(_pe_block, scale=scale),
        grid=grid,
        in_specs=[
            pl.BlockSpec((batch, block_s, d_model), lambda s: (0, s, 0)),
            pl.BlockSpec((block_s, d_model), lambda s: (s, 0)),
        ],
        out_specs=pl.BlockSpec((batch, block_s, d_model), lambda s: (0, s, 0)),
        out_shape=jax.ShapeDtypeStruct(x.shape, x.dtype),
    )(x, pos_table)


_SC_LANES = 16


def _pe_sc_body(x_hbm, pos_hbm, o_hbm, xbuf, pbuf, *, batch, seq_len,
                d_model, base_row, sc_rows, rows_per_worker, chunk_rows,
                scale, num_cores, num_subcores):
    wid = lax.axis_index("s") * num_cores + lax.axis_index("c")
    row0 = base_row + wid * rows_per_worker
    out_row0 = wid * rows_per_worker
    chunk_elems = chunk_rows * d_model
    n_chunks = rows_per_worker // chunk_rows
    n_vec = chunk_elems // _SC_LANES
    unroll = 8

    def chunk_loop(ci, _):
        pos_off = (row0 + ci * chunk_rows) * d_model
        out_base = (out_row0 + ci * chunk_rows) * d_model
        pltpu.sync_copy(pos_hbm.at[pl.ds(pos_off, chunk_elems)], pbuf)

        def batch_loop(b, _):
            x_off = b * (seq_len * d_model) + pos_off
            o_off = b * (sc_rows * d_model) + out_base
            pltpu.sync_copy(x_hbm.at[pl.ds(x_off, chunk_elems)], xbuf)

            def vec_loop(i, _):
                for u in range(unroll):
                    o = (i * unroll + u) * _SC_LANES
                    xv = xbuf[pl.ds(o, _SC_LANES)]
                    pv = pbuf[pl.ds(o, _SC_LANES)]
                    xbuf[pl.ds(o, _SC_LANES)] = xv * scale + pv
                return 0

            lax.fori_loop(0, n_vec // unroll, vec_loop, 0)
            pltpu.sync_copy(xbuf, o_hbm.at[pl.ds(o_off, chunk_elems)])
            return 0

        lax.fori_loop(0, batch, batch_loop, 0)
        return 0

    lax.fori_loop(0, n_chunks, chunk_loop, 0)


def _pe_sc(x, pos_table, base_row, sc_rows, chunk_rows):
    batch, seq_len, d_model = x.shape
    scale = math.sqrt(float(d_model))
    info = plsc.get_sparse_core_info()
    num_cores, num_subcores = info.num_cores, info.num_subcores
    n_workers = num_cores * num_subcores
    rows_per_worker = sc_rows // n_workers
    chunk_rows = min(chunk_rows, rows_per_worker)
    chunk_elems = chunk_rows * d_model

    x_flat = x.reshape(-1)
    pos_flat = pos_table.reshape(-1)

    mesh = plsc.VectorSubcoreMesh(core_axis_name="c", subcore_axis_name="s")
    body = functools.partial(
        _pe_sc_body,
        batch=batch,
        seq_len=seq_len,
        d_model=d_model,
        base_row=base_row,
        sc_rows=sc_rows,
        rows_per_worker=rows_per_worker,
        chunk_rows=chunk_rows,
        scale=scale,
        num_cores=num_cores,
        num_subcores=num_subcores,
    )
    out_flat = pl.kernel(
        body,
        mesh=mesh,
        out_type=jax.ShapeDtypeStruct((batch * sc_rows * d_model,), x.dtype),
        scratch_types=[
            pltpu.VMEM((chunk_elems,), jnp.float32),
            pltpu.VMEM((chunk_elems,), jnp.float32),
        ],
    )(x_flat, pos_flat)
    return out_flat.reshape(batch, sc_rows, d_model)


@functools.partial(jax.jit, static_argnames=("sc_rows", "block_s", "chunk_rows"))
def _pe(x, pos_table, sc_rows=1024, block_s=1024, chunk_rows=64):
    batch, seq_len, d_model = x.shape
    tc_rows = seq_len - sc_rows
    tc_out = _pe_tc(x, pos_table, tc_rows, block_s)
    sc_out = _pe_sc(x, pos_table, tc_rows, sc_rows, chunk_rows)
    return lax.dynamic_update_slice(tc_out, sc_out, (0, tc_rows, 0))


def kernel(x, pos_table):
    return _pe(x, pos_table)


# speedup vs baseline: 3.9748x; 2.4027x over previous
import functools
import math

import jax
import jax.numpy as jnp
from jax import lax
from jax.experimental import pallas as pl
from jax.experimental.pallas import tpu as pltpu
from jax.experimental.pallas import tpu_sc as plsc


def _pe_block(x_ref, pos_ref, o_ref, *, scale):
    o_ref[...] = x_ref[...] * scale + pos_ref[...][None, :, :]


def _pe_tc(x, pos_table, tc_rows, block_s):
    batch, seq_len, d_model = x.shape
    scale = math.sqrt(float(d_model))
    grid = (tc_rows // block_s,)
    return pl.pallas_call(
        functools.partial(_pe_block, scale=scale),
        grid=grid,
        in_specs=[
            pl.BlockSpec((batch, block_s, d_model), lambda s: (0, s, 0)),
            pl.BlockSpec((block_s, d_model), lambda s: (s, 0)),
        ],
        out_specs=pl.BlockSpec((batch, block_s, d_model), lambda s: (0, s, 0)),
        out_shape=jax.ShapeDtypeStruct(x.shape, x.dtype),
    )(x, pos_table)


_SC_LANES = 16


def _pe_sc_body(x_hbm, pos_hbm, o_hbm, xbuf, pbuf, *, batch, d_model,
                base_row, rows_per_worker, chunk_rows, scale, num_cores,
                num_subcores):
    wid = lax.axis_index("s") * num_cores + lax.axis_index("c")
    row0 = base_row + wid * rows_per_worker
    out_row0 = wid * rows_per_worker
    n_chunks = rows_per_worker // chunk_rows
    n_lane_blocks = d_model // _SC_LANES

    def chunk_loop(ci, _):
        src_r = row0 + ci * chunk_rows
        dst_r = out_row0 + ci * chunk_rows
        pltpu.sync_copy(pos_hbm.at[pl.ds(src_r, chunk_rows), :], pbuf)

        def batch_loop(b, _):
            pltpu.sync_copy(x_hbm.at[b, pl.ds(src_r, chunk_rows), :], xbuf)

            def row_loop(r, _):
                for c in range(n_lane_blocks):
                    o = c * _SC_LANES
                    xv = xbuf[r, pl.ds(o, _SC_LANES)]
                    pv = pbuf[r, pl.ds(o, _SC_LANES)]
                    xbuf[r, pl.ds(o, _SC_LANES)] = xv * scale + pv
                return 0

            lax.fori_loop(0, chunk_rows, row_loop, 0)
            pltpu.sync_copy(xbuf, o_hbm.at[b, pl.ds(dst_r, chunk_rows), :])
            return 0

        lax.fori_loop(0, batch, batch_loop, 0)
        return 0

    lax.fori_loop(0, n_chunks, chunk_loop, 0)


def _pe_sc(x, pos_table, base_row, sc_rows, chunk_rows):
    batch, seq_len, d_model = x.shape
    scale = math.sqrt(float(d_model))
    info = plsc.get_sparse_core_info()
    num_cores, num_subcores = info.num_cores, info.num_subcores
    n_workers = num_cores * num_subcores
    rows_per_worker = sc_rows // n_workers
    chunk_rows = min(chunk_rows, rows_per_worker)

    mesh = plsc.VectorSubcoreMesh(core_axis_name="c", subcore_axis_name="s")
    body = functools.partial(
        _pe_sc_body,
        batch=batch,
        d_model=d_model,
        base_row=base_row,
        rows_per_worker=rows_per_worker,
        chunk_rows=chunk_rows,
        scale=scale,
        num_cores=num_cores,
        num_subcores=num_subcores,
    )
    return pl.kernel(
        body,
        mesh=mesh,
        out_type=jax.ShapeDtypeStruct((batch, sc_rows, d_model), x.dtype),
        scratch_types=[
            pltpu.VMEM((chunk_rows, d_model), jnp.float32),
            pltpu.VMEM((chunk_rows, d_model), jnp.float32),
        ],
    )(x, pos_table)


@functools.partial(jax.jit, static_argnames=("sc_rows", "block_s", "chunk_rows"))
def _pe(x, pos_table, sc_rows=1024, block_s=1024, chunk_rows=32):
    batch, seq_len, d_model = x.shape
    tc_rows = seq_len - sc_rows
    tc_out = _pe_tc(x, pos_table, tc_rows, block_s)
    sc_out = _pe_sc(x, pos_table, tc_rows, sc_rows, chunk_rows)
    return lax.dynamic_update_slice(tc_out, sc_out, (0, tc_rows, 0))


def kernel(x, pos_table):
    return _pe(x, pos_table)
